# Initial kernel scaffold; baseline (speedup 1.0000x reference)
import jax
import jax.numpy as jnp
from jax.experimental import pallas as pl


def kernel(x, w1, w2):
    raise NotImplementedError("write your pallas kernel here")


# speedup vs baseline: 1.3884x; 1.3884x over previous
import functools

import jax
import jax.numpy as jnp
from jax.experimental import pallas as pl
from jax.experimental.pallas import tpu as pltpu

_BB = 8


def _se_kernel(x_ref, w1t_ref, w2_ref, o_ref, *, inv_hw):
    x = x_ref[...].astype(jnp.float32)
    avg = jnp.sum(x, axis=-1, keepdims=True) * inv_hw
    w1t = w1t_ref[...].astype(jnp.float32)
    h = jnp.sum(w1t[None] * avg, axis=1, keepdims=True)
    h = jnp.maximum(h, 0.0)
    w2v = w2_ref[...].astype(jnp.float32)
    y = jnp.sum(w2v[None] * h, axis=-1, keepdims=True)
    gate = jax.nn.sigmoid(y)
    o_ref[...] = (x * gate).astype(o_ref.dtype)


def kernel(x, w1, w2):
    B, C, H, W = x.shape
    Cr = w1.shape[0]
    HW = H * W
    x_flat = x.reshape(B, C, HW)

    bb = _BB
    while B % bb:
        bb //= 2
    n_b = B // bb

    block_bytes = bb * C * HW * jnp.dtype(x.dtype).itemsize
    vmem = 4 * block_bytes + bb * C * HW * 4 + 2 * C * Cr * 4 + (256 << 10)

    w1t = jnp.transpose(w1)
    out = pl.pallas_call(
        functools.partial(_se_kernel, inv_hw=1.0 / float(HW)),
        out_shape=jax.ShapeDtypeStruct((B, C, HW), x.dtype),
        grid=(n_b,),
        in_specs=[
            pl.BlockSpec((bb, C, HW), lambda b: (b, 0, 0)),
            pl.BlockSpec((C, Cr), lambda b: (0, 0)),
            pl.BlockSpec((C, Cr), lambda b: (0, 0)),
        ],
        out_specs=pl.BlockSpec((bb, C, HW), lambda b: (b, 0, 0)),
        compiler_params=pltpu.CompilerParams(
            dimension_semantics=("parallel",),
            vmem_limit_bytes=int(min(vmem, 56 << 20)),
        ),
    )(x_flat, w1t, w2)
    return out.reshape(B, C, H, W)


# speedup vs baseline: 1.3933x; 1.0035x over previous
import functools

import jax
import jax.numpy as jnp
from jax.experimental import pallas as pl
from jax.experimental.pallas import tpu as pltpu

_BB = 16


def _se_kernel(x_ref, w1t_ref, w2_ref, o_ref, *, inv_hw):
    x = x_ref[...].astype(jnp.float32)
    avg = jnp.sum(x, axis=-1, keepdims=True) * inv_hw
    w1t = w1t_ref[...].astype(jnp.float32)
    h = jnp.sum(w1t[None] * avg, axis=1, keepdims=True)
    h = jnp.maximum(h, 0.0)
    w2v = w2_ref[...].astype(jnp.float32)
    y = jnp.sum(w2v[None] * h, axis=-1, keepdims=True)
    gate = jax.nn.sigmoid(y)
    o_ref[...] = (x * gate).astype(o_ref.dtype)


def kernel(x, w1, w2):
    B, C, H, W = x.shape
    Cr = w1.shape[0]
    HW = H * W
    x_flat = x.reshape(B, C, HW)

    bb = _BB
    while B % bb:
        bb //= 2
    n_b = B // bb

    block_bytes = bb * C * HW * jnp.dtype(x.dtype).itemsize
    vmem = 4 * block_bytes + bb * C * HW * 4 + 2 * C * Cr * 4 + (256 << 10)

    w1t = jnp.transpose(w1)
    out = pl.pallas_call(
        functools.partial(_se_kernel, inv_hw=1.0 / float(HW)),
        out_shape=jax.ShapeDtypeStruct((B, C, HW), x.dtype),
        grid=(n_b,),
        in_specs=[
            pl.BlockSpec((bb, C, HW), lambda b: (b, 0, 0)),
            pl.BlockSpec((C, Cr), lambda b: (0, 0)),
            pl.BlockSpec((C, Cr), lambda b: (0, 0)),
        ],
        out_specs=pl.BlockSpec((bb, C, HW), lambda b: (b, 0, 0)),
        compiler_params=pltpu.CompilerParams(
            dimension_semantics=("arbitrary",),
            vmem_limit_bytes=int(min(vmem, 56 << 20)),
        ),
    )(x_flat, w1t, w2)
    return out.reshape(B, C, H, W)
